# baseline (speedup 1.0000x reference)
import functools

import jax
import jax.numpy as jnp
from jax.experimental import pallas as pl
from jax.experimental.pallas import tpu as pltpu

_LANES = 128
_VPAD = 32
_TOK_ROWS_ITER = 8


def _embed_tr_kernel(tok_ref, table_ref, out_ref, *, iters):
    table = table_ref[...]
    viota = jax.lax.broadcasted_iota(jnp.int32, (_VPAD, _LANES), 0)
    ntok = _TOK_ROWS_ITER * _LANES

    for c in range(iters):
        r = c * _TOK_ROWS_ITER
        tok = tok_ref[pl.ds(r, _TOK_ROWS_ITER), :]
        cols = [
            (viota == tok[j:j + 1, :]).astype(jnp.bfloat16)
            for j in range(_TOK_ROWS_ITER)
        ]
        onehot_t = jnp.concatenate(cols, axis=1)
        acc = jax.lax.dot_general(
            onehot_t, table,
            dimension_numbers=(((0,), (0,)), ((), ())),
            preferred_element_type=jnp.float32,
        )
        out_ref[pl.ds(c * ntok, ntok), :] = acc


def _encode(tok2d, table_pad):
    n_tok_rows, _ = tok2d.shape
    rows_out = n_tok_rows * _LANES

    tile_tok_rows = 64
    while n_tok_rows % tile_tok_rows:
        tile_tok_rows //= 2
    tile_out_rows = tile_tok_rows * _LANES
    grid = (n_tok_rows // tile_tok_rows,)
    iters = tile_tok_rows // _TOK_ROWS_ITER

    return pl.pallas_call(
        functools.partial(_embed_tr_kernel, iters=iters),
        out_shape=jax.ShapeDtypeStruct((rows_out, _LANES), jnp.float32),
        grid=grid,
        in_specs=[
            pl.BlockSpec((tile_tok_rows, _LANES), lambda i: (i, 0)),
            pl.BlockSpec((_VPAD, _LANES), lambda i: (0, 0)),
        ],
        out_specs=pl.BlockSpec((tile_out_rows, _LANES), lambda i: (i, 0)),
        compiler_params=pltpu.CompilerParams(
            dimension_semantics=("parallel",),
            vmem_limit_bytes=48 * 1024 * 1024,
        ),
    )(tok2d, table_pad)


def kernel(sequence_tokens, embed_table):
    B, S = sequence_tokens.shape
    V, D = embed_table.shape
    N = B * S

    tok = sequence_tokens.reshape(-1).astype(jnp.int32)
    n_pad = -N % (_LANES * _TOK_ROWS_ITER)
    if n_pad:
        tok = jnp.concatenate([tok, jnp.zeros((n_pad,), jnp.int32)])
    tok2d = tok.reshape(-1, _LANES)

    table_pad = jnp.zeros((_VPAD, D), jnp.bfloat16)
    table_pad = table_pad.at[:V].set(embed_table.astype(jnp.bfloat16))

    flat = _encode(tok2d, table_pad)
    if n_pad:
        flat = flat[:N]
    return flat.reshape(B, S, D)


# speedup vs baseline: 1.0998x; 1.0998x over previous
import functools

import jax
import jax.numpy as jnp
from jax.experimental import pallas as pl
from jax.experimental.pallas import tpu as pltpu

_LANES = 128
_VPAD = 32
_TOK_ROWS_ITER = 8


def _embed_tr_kernel(tok_ref, table_ref, out_ref, *, iters):
    table = table_ref[...]
    viota = jax.lax.broadcasted_iota(jnp.int32, (_VPAD, _LANES), 0)
    ntok = _TOK_ROWS_ITER * _LANES

    for c in range(iters):
        r = c * _TOK_ROWS_ITER
        tok = tok_ref[pl.ds(r, _TOK_ROWS_ITER), :]
        cols = [
            (viota == tok[j:j + 1, :]).astype(jnp.bfloat16)
            for j in range(_TOK_ROWS_ITER)
        ]
        onehot_t = jnp.concatenate(cols, axis=1)
        acc = jax.lax.dot_general(
            onehot_t, table,
            dimension_numbers=(((0,), (0,)), ((), ())),
            preferred_element_type=jnp.float32,
        )
        out_ref[pl.ds(c * ntok, ntok), :] = acc


def _encode(tok2d, table_pad):
    n_tok_rows, _ = tok2d.shape
    rows_out = n_tok_rows * _LANES

    tile_tok_rows = 256
    while n_tok_rows % tile_tok_rows:
        tile_tok_rows //= 2
    tile_out_rows = tile_tok_rows * _LANES
    grid = (n_tok_rows // tile_tok_rows,)
    iters = tile_tok_rows // _TOK_ROWS_ITER

    return pl.pallas_call(
        functools.partial(_embed_tr_kernel, iters=iters),
        out_shape=jax.ShapeDtypeStruct((rows_out, _LANES), jnp.float32),
        grid=grid,
        in_specs=[
            pl.BlockSpec((tile_tok_rows, _LANES), lambda i: (i, 0)),
            pl.BlockSpec((_VPAD, _LANES), lambda i: (0, 0)),
        ],
        out_specs=pl.BlockSpec((tile_out_rows, _LANES), lambda i: (i, 0)),
        compiler_params=pltpu.CompilerParams(
            dimension_semantics=("parallel",),
            vmem_limit_bytes=48 * 1024 * 1024,
        ),
    )(tok2d, table_pad)


def kernel(sequence_tokens, embed_table):
    B, S = sequence_tokens.shape
    V, D = embed_table.shape
    N = B * S

    tok = sequence_tokens.reshape(-1).astype(jnp.int32)
    n_pad = -N % (_LANES * _TOK_ROWS_ITER)
    if n_pad:
        tok = jnp.concatenate([tok, jnp.zeros((n_pad,), jnp.int32)])
    tok2d = tok.reshape(-1, _LANES)

    table_pad = jnp.zeros((_VPAD, D), jnp.bfloat16)
    table_pad = table_pad.at[:V].set(embed_table.astype(jnp.bfloat16))

    flat = _encode(tok2d, table_pad)
    if n_pad:
        flat = flat[:N]
    return flat.reshape(B, S, D)
